# baseline (speedup 1.0000x reference)
import functools
import jax
import jax.numpy as jnp
from jax import lax
from jax.experimental import pallas as pl
from jax.experimental.pallas import tpu as pltpu
from jax.experimental.pallas import tpu_sc as plsc

B, E_IN, C, D, K, V = 256, 768, 8, 64, 1024, 256
NW = 32
ROWS_PER_W = (B * C) // NW


def _tc_body(batch_ref, rp_ref, wd_ref, bd_ref, cb_ref, idx_ref):
    c = pl.program_id(0)
    x = batch_ref[...]
    rp = rp_ref[0]
    tq = jnp.dot(x, rp, preferred_element_type=jnp.float32)
    y = jnp.dot(tq, wd_ref[0], preferred_element_type=jnp.float32) + bd_ref[0]
    cb = cb_ref[0]
    xe = lax.dot_general(y, cb, (((1,), (1,)), ((), ())),
                         preferred_element_type=jnp.float32)
    x2 = jnp.sum(y * y, axis=1, keepdims=True)
    e2 = jnp.sum(cb * cb, axis=1)
    dist = -(x2 - 2.0 * xe + e2[None, :])
    m = jnp.max(dist, axis=1, keepdims=True)
    kidx = lax.broadcasted_iota(jnp.int32, (B, K), 1)
    idx = jnp.min(jnp.where(dist == m, kidx, K), axis=1)
    idx_ref[0, 0, :] = idx + c * K


def _sc_gather_mean(gidx_hbm, vflat_hbm, out_hbm, idx_v, rows_v, tmp32, out_v,
                    sem):
    wid = lax.axis_index("s") * 2 + lax.axis_index("c")
    base = wid * ROWS_PER_W
    pltpu.sync_copy(gidx_hbm.at[pl.ds(base, ROWS_PER_W)], idx_v)
    pltpu.async_copy(vflat_hbm.at[idx_v], rows_v, sem).wait()

    lanes = lax.iota(jnp.int32, 16)
    for g in range(ROWS_PER_W // 16):
        grp = jnp.zeros((16,), jnp.float32)
        for l in range(16):
            i = g * 16 + l
            p = rows_v[i, pl.ds(0, 16)]
            for j in range(1, V // 16):
                p = p + rows_v[i, pl.ds(j * 16, 16)]
            for sh in (8, 4, 2, 1):
                tmp32[pl.ds(0, 16)] = p
                tmp32[pl.ds(16, 16)] = p
                p = p + tmp32[pl.ds(sh, 16)]
            grp = jnp.where(lanes == l, p, grp)
        out_v[pl.ds(g * 16, 16)] = grp * (1.0 / V)
    pltpu.sync_copy(out_v, out_hbm.at[pl.ds(base, ROWS_PER_W)])


@jax.jit
def kernel(batch, values, rand_proj, W_in, b_in, codebook):
    W_diag = W_in.reshape(C, D, D).transpose(0, 2, 1)
    b_diag = b_in.reshape(C, 1, D)
    gidx = pl.pallas_call(
        _tc_body,
        grid=(C,),
        in_specs=[
            pl.BlockSpec((B, E_IN), lambda c: (0, 0)),
            pl.BlockSpec((1, E_IN, D), lambda c: (c, 0, 0)),
            pl.BlockSpec((1, D, D), lambda c: (c, 0, 0)),
            pl.BlockSpec((1, 1, D), lambda c: (c, 0, 0)),
            pl.BlockSpec((1, K, D), lambda c: (c, 0, 0)),
        ],
        out_specs=pl.BlockSpec((1, 1, B), lambda c: (c, 0, 0)),
        out_shape=jax.ShapeDtypeStruct((C, 1, B), jnp.int32),
    )(batch, rand_proj, W_diag, b_diag, codebook)

    sc = functools.partial(
        pl.kernel,
        mesh=plsc.VectorSubcoreMesh(core_axis_name="c", subcore_axis_name="s"),
        out_type=jax.ShapeDtypeStruct((C * B,), jnp.float32),
        scratch_types=[
            pltpu.VMEM((ROWS_PER_W,), jnp.int32),
            pltpu.VMEM((ROWS_PER_W, V), jnp.float32),
            pltpu.VMEM((32,), jnp.float32),
            pltpu.VMEM((ROWS_PER_W,), jnp.float32),
            pltpu.SemaphoreType.DMA,
        ],
    )(_sc_gather_mean)
    out_flat = sc(gidx.reshape(C * B), values.reshape(C * K, V))
    return out_flat.reshape(C, B).T
